# baseline (speedup 1.0000x reference)
import functools

import jax
import jax.numpy as jnp
from jax import lax
from jax.experimental import pallas as pl
from jax.experimental.pallas import tpu as pltpu
from jax.experimental.pallas import tpu_sc as plsc

_N_ATOMS = 1_000_000
_N_MOL = 10_000
_W = 8192
_NW = 32
_NFULL = _N_ATOMS // _W
_TAIL_START = _NFULL * _W
_TAIL = _N_ATOMS - _TAIL_START
_WPW = -(-_NFULL // _NW)
_NBUF = 3
_BLK = 262144
_GRID = -(-_N_ATOMS // _BLK)


def _tc_err(fp_t, ft_t):
    def body(fp_ref, ft_ref, err_ref):
        d = fp_ref[...] - ft_ref[...]
        err_ref[...] = jnp.sum(d * d, axis=0, keepdims=True)

    return pl.pallas_call(
        body,
        grid=(_GRID,),
        in_specs=[
            pl.BlockSpec((3, _BLK), lambda i: (0, i)),
            pl.BlockSpec((3, _BLK), lambda i: (0, i)),
        ],
        out_specs=pl.BlockSpec((1, _BLK), lambda i: (0, i)),
        out_shape=jax.ShapeDtypeStruct((1, _N_ATOMS), jnp.float32),
    )(fp_t, ft_t)


def _sc_segsum(err_flat, ids):
    mesh = plsc.VectorSubcoreMesh(core_axis_name="c", subcore_axis_name="s")

    vmem_sets = []
    for _ in range(_NBUF):
        vmem_sets += [
            pltpu.VMEM((_W,), jnp.float32),
            pltpu.VMEM((_W,), jnp.int32),
        ]

    @functools.partial(
        pl.kernel,
        out_type=jax.ShapeDtypeStruct((2, _N_MOL), jnp.float32),
        mesh=mesh,
        compiler_params=pltpu.CompilerParams(needs_layout_passes=False),
        scratch_types=vmem_sets + [
            pltpu.VMEM((_N_MOL,), jnp.float32),
            pltpu.VMEM_SHARED((_N_MOL,), jnp.float32),
        ] + [pltpu.SemaphoreType.DMA] * (2 * _NBUF),
    )
    def seg_kernel(err_hbm, ids_hbm, out_hbm, *refs):
        bufs = [tuple(refs[2 * r:2 * r + 2]) for r in range(_NBUF)]
        stage = refs[2 * _NBUF]
        acc = refs[2 * _NBUF + 1]
        dsems = refs[2 * _NBUF + 2:2 * _NBUF + 2 + _NBUF]
        ssems = refs[2 * _NBUF + 2 + _NBUF:]

        cid = lax.axis_index("c")
        sid = lax.axis_index("s")
        wid = sid * 2 + cid

        @pl.when(sid == 0)
        def _init():
            zeros = jnp.zeros((16,), jnp.float32)

            def zloop(i, carry):
                stage[pl.ds(i * 16, 16)] = zeros
                return carry

            lax.fori_loop(0, _N_MOL // 16, zloop, 0)
            pltpu.sync_copy(stage, acc)

        plsc.subcore_barrier()

        def in_descs(i):
            r = i % _NBUF
            errb, idb = bufs[r]
            a0 = (wid + i * _NW) * _W
            return [
                pltpu.make_async_copy(err_hbm.at[pl.ds(a0, _W)], errb,
                                      dsems[r]),
                pltpu.make_async_copy(ids_hbm.at[pl.ds(a0, _W)], idb,
                                      dsems[r]),
            ]

        def sc_desc(i):
            r = i % _NBUF
            errb, idb = bufs[r]
            return pltpu.make_async_copy(errb, acc.at[idb], ssems[r])

        def valid(i):
            return (wid + i * _NW) < _NFULL

        descs_in = [in_descs(i) for i in range(_WPW)]
        descs_sc = [sc_desc(i) for i in range(_WPW)]

        @pl.when(valid(0))
        def _():
            for d in descs_in[0]:
                d.start()

        for i in range(_WPW):
            @pl.when(valid(i))
            def _(i=i):
                for d in descs_in[i]:
                    d.wait()

            if i >= 2:
                @pl.when(valid(i - 2))
                def _(i=i):
                    descs_sc[i - 2].wait()

            if i + 1 < _WPW:
                @pl.when(valid(i + 1))
                def _(i=i):
                    for d in descs_in[i + 1]:
                        d.start()

            @pl.when(valid(i))
            def _(i=i):
                descs_sc[i].start(add=True)

        for i in (_WPW - 2, _WPW - 1):
            @pl.when(valid(i))
            def _(i=i):
                descs_sc[i].wait()

        @pl.when(wid == _NW - 1)
        def _tail():
            errb, idb = bufs[0]
            pltpu.sync_copy(err_hbm.at[pl.ds(_TAIL_START, _TAIL)],
                            errb.at[pl.ds(0, _TAIL)])
            pltpu.sync_copy(ids_hbm.at[pl.ds(_TAIL_START, _TAIL)],
                            idb.at[pl.ds(0, _TAIL)])
            pltpu.sync_copy(errb.at[pl.ds(0, _TAIL)],
                            acc.at[idb.at[pl.ds(0, _TAIL)]], add=True)

        plsc.subcore_barrier()

        @pl.when(sid == 0)
        def _drain():
            pltpu.sync_copy(acc, stage)
            pltpu.sync_copy(stage, out_hbm.at[cid])

    return seg_kernel(err_flat, ids)


def _finish(partial, counts2d, ep2d, et2d):
    def body(p_ref, cnt_ref, ep_ref, et_ref, comb_ref, el_ref, fl_ref):
        psum = p_ref[0:1, :] + p_ref[1:2, :]
        scale = 1.0 / (3.0 * cnt_ref[...].astype(jnp.float32))
        fl = psum * scale
        el = (ep_ref[...] - et_ref[...]) ** 2
        fl_ref[...] = fl
        el_ref[...] = el
        comb_ref[0, 0] = (jnp.sum(el) + jnp.sum(fl)) / jnp.float32(_N_MOL)

    return pl.pallas_call(
        body,
        out_shape=(
            jax.ShapeDtypeStruct((1, 1), jnp.float32),
            jax.ShapeDtypeStruct((1, _N_MOL), jnp.float32),
            jax.ShapeDtypeStruct((1, _N_MOL), jnp.float32),
        ),
        out_specs=(
            pl.BlockSpec(memory_space=pltpu.SMEM),
            pl.BlockSpec(memory_space=pltpu.VMEM),
            pl.BlockSpec(memory_space=pltpu.VMEM),
        ),
    )(partial, counts2d, ep2d, et2d)


def kernel(F_predict, F_true, E_predict, E_true, atomic_subsystem_indices,
           atomic_subsystem_counts):
    err = _tc_err(F_predict.T, F_true.T).reshape(_N_ATOMS)
    partial = _sc_segsum(err, atomic_subsystem_indices)
    comb, el, fl = _finish(
        partial,
        atomic_subsystem_counts.reshape(1, _N_MOL),
        E_predict.reshape(1, _N_MOL),
        E_true.reshape(1, _N_MOL),
    )
    return comb[0, 0], el.reshape(_N_MOL), fl.reshape(_N_MOL)


# speedup vs baseline: 1.0092x; 1.0092x over previous
import functools

import jax
import jax.numpy as jnp
from jax import lax
from jax.experimental import pallas as pl
from jax.experimental.pallas import tpu as pltpu
from jax.experimental.pallas import tpu_sc as plsc

_N_ATOMS = 1_000_000
_N_MOL = 10_000
_W = 4096
_NW = 32
_NFULL = _N_ATOMS // _W
_TAIL_START = _NFULL * _W
_TAIL = _N_ATOMS - _TAIL_START
_WPW = -(-_NFULL // _NW)
_NBUF = 3
_BLK = 262144
_GRID = -(-_N_ATOMS // _BLK)


def _tc_err(fp_t, ft_t):
    def body(fp_ref, ft_ref, err_ref):
        d = fp_ref[...] - ft_ref[...]
        err_ref[...] = jnp.sum(d * d, axis=0, keepdims=True)

    return pl.pallas_call(
        body,
        grid=(_GRID,),
        in_specs=[
            pl.BlockSpec((3, _BLK), lambda i: (0, i)),
            pl.BlockSpec((3, _BLK), lambda i: (0, i)),
        ],
        out_specs=pl.BlockSpec((1, _BLK), lambda i: (0, i)),
        out_shape=jax.ShapeDtypeStruct((1, _N_ATOMS), jnp.float32),
    )(fp_t, ft_t)


def _sc_segsum(err_flat, ids):
    mesh = plsc.VectorSubcoreMesh(core_axis_name="c", subcore_axis_name="s")

    vmem_sets = []
    for _ in range(_NBUF):
        vmem_sets += [
            pltpu.VMEM((_W,), jnp.float32),
            pltpu.VMEM((_W,), jnp.int32),
        ]

    @functools.partial(
        pl.kernel,
        out_type=jax.ShapeDtypeStruct((2, _N_MOL), jnp.float32),
        mesh=mesh,
        compiler_params=pltpu.CompilerParams(needs_layout_passes=False),
        scratch_types=vmem_sets + [
            pltpu.VMEM((_N_MOL,), jnp.float32),
            pltpu.VMEM_SHARED((_N_MOL,), jnp.float32),
        ] + [pltpu.SemaphoreType.DMA] * (2 * _NBUF),
    )
    def seg_kernel(err_hbm, ids_hbm, out_hbm, *refs):
        bufs = [tuple(refs[2 * r:2 * r + 2]) for r in range(_NBUF)]
        stage = refs[2 * _NBUF]
        acc = refs[2 * _NBUF + 1]
        dsems = refs[2 * _NBUF + 2:2 * _NBUF + 2 + _NBUF]
        ssems = refs[2 * _NBUF + 2 + _NBUF:]

        cid = lax.axis_index("c")
        sid = lax.axis_index("s")
        wid = sid * 2 + cid

        @pl.when(sid == 0)
        def _init():
            zeros = jnp.zeros((16,), jnp.float32)

            def zloop(i, carry):
                stage[pl.ds(i * 16, 16)] = zeros
                return carry

            lax.fori_loop(0, _N_MOL // 16, zloop, 0)
            pltpu.sync_copy(stage, acc)

        plsc.subcore_barrier()

        def in_descs(i):
            r = i % _NBUF
            errb, idb = bufs[r]
            a0 = (wid + i * _NW) * _W
            return [
                pltpu.make_async_copy(err_hbm.at[pl.ds(a0, _W)], errb,
                                      dsems[r]),
                pltpu.make_async_copy(ids_hbm.at[pl.ds(a0, _W)], idb,
                                      dsems[r]),
            ]

        def sc_desc(i):
            r = i % _NBUF
            errb, idb = bufs[r]
            return pltpu.make_async_copy(errb, acc.at[idb], ssems[r])

        def valid(i):
            return (wid + i * _NW) < _NFULL

        descs_in = [in_descs(i) for i in range(_WPW)]
        descs_sc = [sc_desc(i) for i in range(_WPW)]

        @pl.when(valid(0))
        def _():
            for d in descs_in[0]:
                d.start()

        for i in range(_WPW):
            @pl.when(valid(i))
            def _(i=i):
                for d in descs_in[i]:
                    d.wait()

            if i >= 2:
                @pl.when(valid(i - 2))
                def _(i=i):
                    descs_sc[i - 2].wait()

            if i + 1 < _WPW:
                @pl.when(valid(i + 1))
                def _(i=i):
                    for d in descs_in[i + 1]:
                        d.start()

            @pl.when(valid(i))
            def _(i=i):
                descs_sc[i].start(add=True)

        for i in (_WPW - 2, _WPW - 1):
            @pl.when(valid(i))
            def _(i=i):
                descs_sc[i].wait()

        @pl.when(wid == _NW - 1)
        def _tail():
            errb, idb = bufs[0]
            pltpu.sync_copy(err_hbm.at[pl.ds(_TAIL_START, _TAIL)],
                            errb.at[pl.ds(0, _TAIL)])
            pltpu.sync_copy(ids_hbm.at[pl.ds(_TAIL_START, _TAIL)],
                            idb.at[pl.ds(0, _TAIL)])
            pltpu.sync_copy(errb.at[pl.ds(0, _TAIL)],
                            acc.at[idb.at[pl.ds(0, _TAIL)]], add=True)

        plsc.subcore_barrier()

        @pl.when(sid == 0)
        def _drain():
            pltpu.sync_copy(acc, stage)
            pltpu.sync_copy(stage, out_hbm.at[cid])

    return seg_kernel(err_flat, ids)


def _finish(partial, counts2d, ep2d, et2d):
    def body(p_ref, cnt_ref, ep_ref, et_ref, comb_ref, el_ref, fl_ref):
        psum = p_ref[0:1, :] + p_ref[1:2, :]
        scale = 1.0 / (3.0 * cnt_ref[...].astype(jnp.float32))
        fl = psum * scale
        el = (ep_ref[...] - et_ref[...]) ** 2
        fl_ref[...] = fl
        el_ref[...] = el
        comb_ref[0, 0] = (jnp.sum(el) + jnp.sum(fl)) / jnp.float32(_N_MOL)

    return pl.pallas_call(
        body,
        out_shape=(
            jax.ShapeDtypeStruct((1, 1), jnp.float32),
            jax.ShapeDtypeStruct((1, _N_MOL), jnp.float32),
            jax.ShapeDtypeStruct((1, _N_MOL), jnp.float32),
        ),
        out_specs=(
            pl.BlockSpec(memory_space=pltpu.SMEM),
            pl.BlockSpec(memory_space=pltpu.VMEM),
            pl.BlockSpec(memory_space=pltpu.VMEM),
        ),
    )(partial, counts2d, ep2d, et2d)


def kernel(F_predict, F_true, E_predict, E_true, atomic_subsystem_indices,
           atomic_subsystem_counts):
    err = _tc_err(F_predict.T, F_true.T).reshape(_N_ATOMS)
    partial = _sc_segsum(err, atomic_subsystem_indices)
    comb, el, fl = _finish(
        partial,
        atomic_subsystem_counts.reshape(1, _N_MOL),
        E_predict.reshape(1, _N_MOL),
        E_true.reshape(1, _N_MOL),
    )
    return comb[0, 0], el.reshape(_N_MOL), fl.reshape(_N_MOL)
